# baseline (speedup 1.0000x reference)
import jax
import jax.numpy as jnp
from jax import lax
from jax.experimental import pallas as pl
from jax.experimental.pallas import tpu as pltpu
from jax.experimental.pallas import tpu_sc as plsc

MAX_VALUE = 100000
D = 128
B = 1024
H = W = 16
HW = H * W
HW2 = HW // 2
C = 3
CHUNK = 64
NCHUNK = HW // CHUNK
LANES = 16
NC, NS = 2, 16
NWORK = NC * NS
BPW = B // NWORK
JGROUPS = C * HW // LANES


def _sc_body(emb_hbm, idx_hbm, out_hbm,
             idxraw0, idxraw1, idxadj0, idxadj1, rows0, rows1, outt0, outt1,
             sem_g0, sem_g1, sem_idx, sem_out):
    sem_g = (sem_g0, sem_g1)
    idxraw = (idxraw0, idxraw1)
    idxadj = (idxadj0, idxadj1)
    rows = (rows0, rows1)
    outt = (outt0, outt1)
    wid = lax.axis_index("s") * NC + lax.axis_index("c")
    iota = lax.iota(jnp.int32, LANES)
    row_ids = [iota + 16 * g for g in range(D // LANES)]
    base = wid * BPW

    def adjust(ib):
        for j in range(JGROUPS):
            c = j // (HW // LANES)
            val = idxraw[ib][pl.ds(LANES * j, LANES)] + jnp.int32(c * MAX_VALUE)
            idxadj[ib][j // 8, pl.ds(LANES * (j % 8), LANES)] = val

    def chunk_idx_ref(ib, ck, c):
        return idxadj[ib].at[2 * c + ck // 2, pl.ds(CHUNK * (ck % 2), CHUNK)]

    def issue_gathers(ib, ck, rb):
        for c in range(C):
            pltpu.async_copy(emb_hbm.at[chunk_idx_ref(ib, ck, c)],
                             rows[rb].at[c], sem_g[rb])

    def wait_gathers(rb):
        for c in range(C):
            pltpu.make_async_copy(emb_hbm.at[chunk_idx_ref(0, 0, 0)],
                                  rows[rb].at[c], sem_g[rb]).wait()

    def compute_chunk(rb, ck):
        @plsc.parallel_loop(0, CHUNK, unroll=4)
        def _(p):
            col = jnp.full((LANES,), CHUNK * (ck % 2) + p, dtype=jnp.int32)
            for g in range(D // LANES):
                v = (rows[rb][0, p, pl.ds(16 * g, LANES)]
                     + rows[rb][1, p, pl.ds(16 * g, LANES)]
                     + rows[rb][2, p, pl.ds(16 * g, LANES)])
                plsc.store_scatter(outt[ck // 2], [row_ids[g], col], v)

    def out_write_copy(h, gb):
        return pltpu.make_async_copy(
            outt[h].at[:, pl.ds(0, HW2)],
            out_hbm.at[gb, :, pl.ds(h * HW2, HW2)], sem_out)

    def emit_batch(gb, ob, guard_next, guard_prev):
        nxt = 1 - ob

        def maybe(pred, fn):
            def run():
                fn()
            if pred is None:
                run()
            else:
                pl.when(pred)(run)

        maybe(guard_next, lambda: pltpu.async_copy(
            idx_hbm.at[gb + 1], idxraw[nxt], sem_idx))

        for ck in range(NCHUNK):
            rb = ck % 2
            if ck % 2 == 0:
                maybe(guard_prev, lambda: out_write_copy(ck // 2, gb - 1).wait())
            if ck < NCHUNK - 1:
                issue_gathers(ob, ck + 1, (ck + 1) % 2)
            else:
                def _next_batch_head():
                    pltpu.make_async_copy(idx_hbm.at[gb + 1],
                                          idxraw[nxt], sem_idx).wait()
                    adjust(nxt)
                    issue_gathers(nxt, 0, 0)
                maybe(guard_next, _next_batch_head)
            wait_gathers(rb)
            compute_chunk(rb, ck)
            if ck % 2 == 1:
                out_write_copy(ck // 2, gb).start()

    pltpu.sync_copy(idx_hbm.at[base], idxraw[0])
    adjust(0)
    issue_gathers(0, 0, 0)

    NPAIR = BPW // 2

    def per_pair(i, _):
        emit_batch(base + 2 * i, 0, None, i > 0)
        emit_batch(base + 2 * i + 1, 1, i < NPAIR - 1, None)
        return _

    lax.fori_loop(0, NPAIR, per_pair, None)
    out_write_copy(0, base + BPW - 1).wait()
    out_write_copy(1, base + BPW - 1).wait()


def kernel(inputs, embedding):
    idx = inputs.reshape(B, C * HW).astype(jnp.int32)
    emb = embedding.astype(jnp.float32)

    mesh = plsc.VectorSubcoreMesh(
        core_axis_name="c", subcore_axis_name="s", num_cores=NC, num_subcores=NS
    )
    run = pl.kernel(
        _sc_body,
        out_type=jax.ShapeDtypeStruct((B, D, HW), jnp.float32),
        mesh=mesh,
        scratch_types=[
            pltpu.VMEM((C * HW,), jnp.int32),
            pltpu.VMEM((C * HW,), jnp.int32),
            pltpu.VMEM((JGROUPS // 8, 128), jnp.int32),
            pltpu.VMEM((JGROUPS // 8, 128), jnp.int32),
            pltpu.VMEM((C, CHUNK, D), jnp.float32),
            pltpu.VMEM((C, CHUNK, D), jnp.float32),
            pltpu.VMEM((D, HW2 + 1), jnp.float32),
            pltpu.VMEM((D, HW2 + 1), jnp.float32),
            pltpu.SemaphoreType.DMA,
            pltpu.SemaphoreType.DMA,
            pltpu.SemaphoreType.DMA,
            pltpu.SemaphoreType.DMA,
        ],
        compiler_params=pltpu.CompilerParams(needs_layout_passes=False),
    )
    out = run(emb, idx)
    return out.reshape(B, D, H, W)


# speedup vs baseline: 1.3943x; 1.3943x over previous
import jax
import jax.numpy as jnp
from jax import lax
from jax.experimental import pallas as pl
from jax.experimental.pallas import tpu as pltpu
from jax.experimental.pallas import tpu_sc as plsc

MAX_VALUE = 100000
D = 128
B = 1024
H = W = 16
HW = H * W
HW2 = HW // 2
C = 3
CHUNK = 64
NCHUNK = HW // CHUNK
LANES = 16
NC, NS = 2, 16
NWORK = NC * NS
BPW = B // NWORK
JGROUPS = C * HW // LANES


def _sc_body(emb_hbm, idx_hbm, out_hbm,
             idxraw0, idxraw1, idxadj0, idxadj1, rows0, rows1, outt0, outt1,
             sem_g0, sem_g1, sem_idx, sem_out):
    sem_g = (sem_g0, sem_g1)
    idxraw = (idxraw0, idxraw1)
    idxadj = (idxadj0, idxadj1)
    rows = (rows0, rows1)
    outt = (outt0, outt1)
    wid = lax.axis_index("s") * NC + lax.axis_index("c")
    base = wid * BPW

    def adjust(ib):
        for j in range(JGROUPS):
            c = j // (HW // LANES)
            val = idxraw[ib][pl.ds(LANES * j, LANES)] + jnp.int32(c * MAX_VALUE)
            idxadj[ib][j // 8, pl.ds(LANES * (j % 8), LANES)] = val

    def chunk_idx_ref(ib, ck, c):
        return idxadj[ib].at[2 * c + ck // 2, pl.ds(CHUNK * (ck % 2), CHUNK)]

    def issue_gathers(ib, ck, rb):
        for c in range(C):
            pltpu.async_copy(emb_hbm.at[chunk_idx_ref(ib, ck, c)],
                             rows[rb].at[c], sem_g[rb])

    def wait_gathers(rb):
        for c in range(C):
            pltpu.make_async_copy(emb_hbm.at[chunk_idx_ref(0, 0, 0)],
                                  rows[rb].at[c], sem_g[rb]).wait()

    def compute_chunk(rb, ck):
        @plsc.parallel_loop(0, CHUNK, unroll=4)
        def _(p):
            for g in range(D // LANES):
                v = (rows[rb][0, p, pl.ds(16 * g, LANES)]
                     + rows[rb][1, p, pl.ds(16 * g, LANES)]
                     + rows[rb][2, p, pl.ds(16 * g, LANES)])
                outt[ck // 2][CHUNK * (ck % 2) + p, pl.ds(16 * g, LANES)] = v

    def out_write_copy(h, gb):
        return pltpu.make_async_copy(
            outt[h], out_hbm.at[gb, pl.ds(h * HW2, HW2), :], sem_out)

    def emit_batch(gb, ob, guard_next, guard_prev):
        nxt = 1 - ob

        def maybe(pred, fn):
            def run():
                fn()
            if pred is None:
                run()
            else:
                pl.when(pred)(run)

        maybe(guard_next, lambda: pltpu.async_copy(
            idx_hbm.at[gb + 1], idxraw[nxt], sem_idx))

        for ck in range(NCHUNK):
            rb = ck % 2
            if ck % 2 == 0:
                maybe(guard_prev, lambda: out_write_copy(ck // 2, gb - 1).wait())
            if ck < NCHUNK - 1:
                issue_gathers(ob, ck + 1, (ck + 1) % 2)
            else:
                def _next_batch_head():
                    pltpu.make_async_copy(idx_hbm.at[gb + 1],
                                          idxraw[nxt], sem_idx).wait()
                    adjust(nxt)
                    issue_gathers(nxt, 0, 0)
                maybe(guard_next, _next_batch_head)
            wait_gathers(rb)
            compute_chunk(rb, ck)
            if ck % 2 == 1:
                out_write_copy(ck // 2, gb).start()

    pltpu.sync_copy(idx_hbm.at[base], idxraw[0])
    adjust(0)
    issue_gathers(0, 0, 0)

    NPAIR = BPW // 2

    def per_pair(i, _):
        emit_batch(base + 2 * i, 0, None, i > 0)
        emit_batch(base + 2 * i + 1, 1, i < NPAIR - 1, None)
        return _

    lax.fori_loop(0, NPAIR, per_pair, None)
    out_write_copy(0, base + BPW - 1).wait()
    out_write_copy(1, base + BPW - 1).wait()


def kernel(inputs, embedding):
    idx = inputs.reshape(B, C * HW).astype(jnp.int32)
    emb = embedding.astype(jnp.float32)

    mesh = plsc.VectorSubcoreMesh(
        core_axis_name="c", subcore_axis_name="s", num_cores=NC, num_subcores=NS
    )
    run = pl.kernel(
        _sc_body,
        out_type=jax.ShapeDtypeStruct((B, HW, D), jnp.float32),
        mesh=mesh,
        scratch_types=[
            pltpu.VMEM((C * HW,), jnp.int32),
            pltpu.VMEM((C * HW,), jnp.int32),
            pltpu.VMEM((JGROUPS // 8, 128), jnp.int32),
            pltpu.VMEM((JGROUPS // 8, 128), jnp.int32),
            pltpu.VMEM((C, CHUNK, D), jnp.float32),
            pltpu.VMEM((C, CHUNK, D), jnp.float32),
            pltpu.VMEM((HW2, D), jnp.float32),
            pltpu.VMEM((HW2, D), jnp.float32),
            pltpu.SemaphoreType.DMA,
            pltpu.SemaphoreType.DMA,
            pltpu.SemaphoreType.DMA,
            pltpu.SemaphoreType.DMA,
        ],
        compiler_params=pltpu.CompilerParams(needs_layout_passes=False),
    )
    out_pm = run(emb, idx)

    TB = 8

    def _tc_transpose(x_ref, o_ref):
        o_ref[...] = jnp.swapaxes(x_ref[...], 1, 2)

    out = pl.pallas_call(
        _tc_transpose,
        grid=(B // TB,),
        in_specs=[pl.BlockSpec((TB, HW, D), lambda i: (i, 0, 0))],
        out_specs=pl.BlockSpec((TB, D, HW), lambda i: (i, 0, 0)),
        out_shape=jax.ShapeDtypeStruct((B, D, HW), jnp.float32),
    )(out_pm)
    return out.reshape(B, D, H, W)
